# Initial kernel scaffold; baseline (speedup 1.0000x reference)
import jax
import jax.numpy as jnp
from jax.experimental import pallas as pl


def kernel(x, edge_index, W_gat1, att1_src, att1_dst, b_gat1, W_gcn1, b_gcn1, W_a, b_a, W_l, b_l, W_d1, b_d1, W_d2, b_d2, W_gcn2, b_gcn2, W_gat2, att2_src, att2_dst, b_gat2):
    raise NotImplementedError("write your pallas kernel here")


# speedup vs baseline: 7.9357x; 7.9357x over previous
import functools

import jax
import jax.numpy as jnp
from jax import lax
from jax.experimental import pallas as pl
from jax.experimental.pallas import tpu as pltpu
from jax.experimental.pallas import tpu_sc as plsc

N = 10000
NPAD = 10240
D_IN = 512
HEADS = 2

NW = 32
S = 42
B = 128
EPAD = NW * S * B
RPT = NPAD // 16

@functools.cache
def _mesh():
  return plsc.VectorSubcoreMesh(
      core_axis_name="c", subcore_axis_name="s", num_cores=2, num_subcores=16)


def _cdiv(a, b):
  return (a + b - 1) // b


def _mm(a, w, b=None, relu=False, deg=None, chunked=False):
  m, k = a.shape
  _, n = w.shape
  bm = 256
  bn = min(n, 128)
  gm, gn = _cdiv(m, bm), n // bn

  in_specs = [
      pl.BlockSpec((bm, k), lambda i, j: (i, 0)),
      pl.BlockSpec((k, bn), lambda i, j: (0, j)),
  ]
  args = [a, w]
  if b is not None:
    in_specs.append(pl.BlockSpec((1, bn), lambda i, j: (0, j)))
    args.append(b.reshape(1, n))
  if deg is not None:
    in_specs.append(pl.BlockSpec((2, bm, 128), lambda i, j: (0, i, 0)))
    args.append(deg)

  if chunked:
    out_shape = jax.ShapeDtypeStruct((gn, m, bn), jnp.float32)
    out_spec = pl.BlockSpec((1, bm, bn), lambda i, j: (j, i, 0))
  else:
    out_shape = jax.ShapeDtypeStruct((m, n), jnp.float32)
    out_spec = pl.BlockSpec((bm, bn), lambda i, j: (i, j))

  def body(*refs):
    a_ref, w_ref = refs[0], refs[1]
    pos = 2
    b_ref = None
    deg_ref = None
    if b is not None:
      b_ref = refs[pos]; pos += 1
    if deg is not None:
      deg_ref = refs[pos]; pos += 1
    o_ref = refs[pos]
    acc = jnp.dot(a_ref[...].astype(jnp.bfloat16),
                  w_ref[...].astype(jnp.bfloat16),
                  preferred_element_type=jnp.float32)
    if b_ref is not None:
      acc = acc + b_ref[...]
    if deg_ref is not None:
      dp = deg_ref[...]
      degv = dp[0, :, 0] + dp[1, :, 0]
      dinv = jnp.where(degv > 0, 1.0 / jnp.sqrt(degv), 0.0)
      acc = acc * dinv[:, None]
    if relu:
      acc = jnp.maximum(acc, 0.0)
    if chunked:
      o_ref[...] = acc[None]
    else:
      o_ref[...] = acc

  return pl.pallas_call(
      body, grid=(gm, gn), in_specs=in_specs, out_specs=out_spec,
      out_shape=out_shape)(*args)


def _asad(hc, a1):
  c = hc.shape[0]
  gm = _cdiv(N, 256)

  def body(h_ref, a_ref, o_ref):
    ci = pl.program_id(1)

    @pl.when(ci == 0)
    def _():
      o_ref[...] = jnp.zeros((256, 128), jnp.float32)

    o_ref[...] += jnp.dot(h_ref[0], a_ref[...],
                          preferred_element_type=jnp.float32,
                          precision=lax.Precision.HIGHEST)

  return pl.pallas_call(
      body, grid=(gm, c),
      in_specs=[
          pl.BlockSpec((1, 256, 128), lambda i, ci: (ci, i, 0)),
          pl.BlockSpec((128, 128), lambda i, ci: (ci, 0)),
      ],
      out_specs=pl.BlockSpec((256, 128), lambda i, ci: (i, 0)),
      out_shape=jax.ShapeDtypeStruct((N, 128), jnp.float32))(hc, a1)


def _transpose128(a):
  gm = NPAD // 256

  def body(a_ref, o_ref):
    o_ref[...] = a_ref[...].T

  return pl.pallas_call(
      body, grid=(gm,),
      in_specs=[pl.BlockSpec((256, 128), lambda i: (i, 0))],
      out_specs=pl.BlockSpec((128, 256), lambda i: (0, i)),
      out_shape=jax.ShapeDtypeStruct((128, NPAD), jnp.float32))(a)


def _colmax(a):
  gm = _cdiv(N, 256)

  def body(a_ref, o_ref):
    i = pl.program_id(0)

    @pl.when(i == 0)
    def _():
      o_ref[...] = jnp.full((8, 128), -jnp.inf, jnp.float32)

    rid = lax.broadcasted_iota(jnp.int32, (256, 1), 0) + i * 256
    av = jnp.where(rid < N, a_ref[...], -jnp.inf)
    mv = jnp.max(av, axis=0)[None]
    o_ref[...] = jnp.maximum(o_ref[...], jnp.broadcast_to(mv, (8, 128)))

  return pl.pallas_call(
      body, grid=(gm,),
      in_specs=[pl.BlockSpec((256, 128), lambda i: (i, 0))],
      out_specs=pl.BlockSpec((8, 128), lambda i: (0, 0)),
      out_shape=jax.ShapeDtypeStruct((8, 128), jnp.float32))(a)


def _combine_gat(p, den, b, relu):
  c = p.shape[1]
  n_out = c * 128
  gm = _cdiv(N, 256)

  def body(p_ref, d_ref, b_ref, o_ref):
    j = pl.program_id(1)
    hidx = j * HEADS // c
    s = p_ref[0, 0] + p_ref[1, 0]
    d16 = d_ref[0] + d_ref[1]
    oh = (lax.broadcasted_iota(jnp.int32, (1, 128), 1) == hidx)
    denv = jnp.sum(d16 * oh.astype(jnp.float32), axis=1)
    acc = s / (denv[:, None] + 1e-16) + b_ref[...]
    if relu:
      acc = jnp.maximum(acc, 0.0)
    o_ref[...] = acc

  return pl.pallas_call(
      body, grid=(gm, c),
      in_specs=[
          pl.BlockSpec((2, 1, 256, 128), lambda i, j: (0, j, i, 0)),
          pl.BlockSpec((2, 256, 128), lambda i, j: (0, i, 0)),
          pl.BlockSpec((1, 128), lambda i, j: (0, j)),
      ],
      out_specs=pl.BlockSpec((256, 128), lambda i, j: (i, j)),
      out_shape=jax.ShapeDtypeStruct((N, n_out), jnp.float32),
  )(p, den, b.reshape(1, n_out))


def _combine_gcn(p, deg, b, relu):
  c = p.shape[1]
  n_out = c * 128
  gm = _cdiv(N, 256)

  def body(p_ref, d_ref, b_ref, o_ref):
    s = p_ref[0, 0] + p_ref[1, 0]
    degv = d_ref[0, :, 0] + d_ref[1, :, 0]
    dinv = jnp.where(degv > 0, 1.0 / jnp.sqrt(degv), 0.0)
    acc = s * dinv[:, None] + b_ref[...]
    if relu:
      acc = jnp.maximum(acc, 0.0)
    o_ref[...] = acc

  return pl.pallas_call(
      body, grid=(gm, c),
      in_specs=[
          pl.BlockSpec((2, 1, 256, 128), lambda i, j: (0, j, i, 0)),
          pl.BlockSpec((2, 256, 128), lambda i, j: (0, i, 0)),
          pl.BlockSpec((1, 128), lambda i, j: (0, j)),
      ],
      out_specs=pl.BlockSpec((256, 128), lambda i, j: (i, j)),
      out_shape=jax.ShapeDtypeStruct((N, n_out), jnp.float32),
  )(p, deg, b.reshape(1, n_out))


def _cdist(r):
  gm = _cdiv(N, 256)

  def body(a_ref, b_ref, o_ref):
    av = a_ref[...]
    bv = b_ref[...]
    g = lax.dot_general(av, bv, (((1,), (1,)), ((), ())),
                        preferred_element_type=jnp.float32,
                        precision=lax.Precision.HIGHEST)
    sa = jnp.sum(av * av, axis=1)
    sb = jnp.sum(bv * bv, axis=1)
    d2 = sa[:, None] + sb[None, :] - 2.0 * g
    d2 = jnp.maximum(d2, 0.0)
    dist = jnp.sqrt(jnp.where(d2 > 0, d2, 1.0))
    o_ref[...] = jnp.where(d2 > 0, dist, 0.0)

  return pl.pallas_call(
      body, grid=(gm, gm),
      in_specs=[
          pl.BlockSpec((256, 1024), lambda i, j: (i, 0)),
          pl.BlockSpec((256, 1024), lambda i, j: (j, 0)),
      ],
      out_specs=pl.BlockSpec((256, 256), lambda i, j: (i, j)),
      out_shape=jax.ShapeDtypeStruct((N, N), jnp.float32))(r, r)


def _sc_deg(dstp):
  @functools.partial(
      pl.kernel,
      out_type=jax.ShapeDtypeStruct((2, NPAD, 128), jnp.float32),
      mesh=_mesh(),
      scratch_types=[
          pltpu.VMEM((S, B), jnp.int32),
          pltpu.VMEM((B,), jnp.int32),
          pltpu.VMEM((B, 128), jnp.float32),
          pltpu.VMEM((B, 128), jnp.float32),
          pltpu.VMEM_SHARED((NPAD, 128), jnp.float32),
      ],
  )
  def k(dst_hbm, out_hbm, didx, dadj, ones_b, zeros_b, slab):
    cid = lax.axis_index("c")
    sid = lax.axis_index("s")
    wid = sid * 2 + cid
    base = sid * RPT
    pltpu.sync_copy(dst_hbm.at[wid], didx)
    oh = jnp.where(lax.iota(jnp.int32, 16) == 0, 1.0, 0.0)
    zv = jnp.zeros((16,), jnp.float32)
    for r in range(B):
      for kk in range(8):
        ones_b[r, pl.ds(kk * 16, 16)] = oh if kk == 0 else zv
        zeros_b[r, pl.ds(kk * 16, 16)] = zv
    for t in range(RPT // B):
      pltpu.sync_copy(zeros_b, slab.at[pl.ds(base + t * B, B)])
    plsc.subcore_barrier()

    def step(j, carry):
      for kk in range(8):
        dadj[pl.ds(kk * 16, 16)] = didx[j, pl.ds(kk * 16, 16)]
      pltpu.sync_copy(ones_b, slab.at[dadj], add=True)
      return carry

    lax.fori_loop(0, S, step, 0)
    plsc.subcore_barrier()
    pltpu.sync_copy(slab.at[pl.ds(base, RPT)],
                    out_hbm.at[cid, pl.ds(base, RPT)])

  return k(dstp)


def _sc_gat_w(srcp, dstp, tabs, mmax):
  @functools.partial(
      pl.kernel,
      out_type=[
          jax.ShapeDtypeStruct((NW, HEADS, S, B), jnp.float32),
          jax.ShapeDtypeStruct((2, NPAD, 128), jnp.float32),
      ],
      mesh=_mesh(),
      scratch_types=[
          pltpu.VMEM((S, B), jnp.int32),
          pltpu.VMEM((S, B), jnp.int32),
          pltpu.VMEM((B,), jnp.int32),
          pltpu.VMEM((B,), jnp.int32),
          pltpu.VMEM((8, 128), jnp.float32),
          pltpu.VMEM((B,), jnp.float32),
          pltpu.VMEM((B,), jnp.float32),
          pltpu.VMEM((B,), jnp.float32),
          pltpu.VMEM((B,), jnp.float32),
          pltpu.VMEM((HEADS, B), jnp.float32),
          pltpu.VMEM((B, 128), jnp.float32),
          pltpu.VMEM((B, 128), jnp.float32),
          pltpu.SemaphoreType.DMA,
          pltpu.VMEM_SHARED((NPAD, 128), jnp.float32),
      ],
  )
  def k(src_hbm, dst_hbm, as0_hbm, as1_hbm, ad0_hbm, ad1_hbm, m_hbm,
        w_hbm, den_hbm,
        sidx, didx, sadjb, dadjb, m_v, a0b, a1b, d0b, d1b, wbuf, dbuf,
        zeros_b, sem, slab):
    cid = lax.axis_index("c")
    sid = lax.axis_index("s")
    wid = sid * 2 + cid
    base = sid * RPT
    pltpu.sync_copy(src_hbm.at[wid], sidx)
    pltpu.sync_copy(dst_hbm.at[wid], didx)
    pltpu.sync_copy(m_hbm, m_v)
    mv16 = m_v[0, pl.ds(0, 16)]
    s0 = mv16[0] + mv16[2]
    s1 = mv16[1] + mv16[3]
    mh0 = jnp.maximum(s0, 0.2 * s0)
    mh1 = jnp.maximum(s1, 0.2 * s1)
    zv = jnp.zeros((16,), jnp.float32)
    oh0 = jnp.where(lax.iota(jnp.int32, 16) == 0, 1.0, 0.0)
    oh1 = jnp.where(lax.iota(jnp.int32, 16) == 1, 1.0, 0.0)

    for r in range(B):
      for kk in range(8):
        zeros_b[r, pl.ds(kk * 16, 16)] = zv
        dbuf[r, pl.ds(kk * 16, 16)] = zv
    for t in range(RPT // B):
      pltpu.sync_copy(zeros_b, slab.at[pl.ds(base + t * B, B)])
    plsc.subcore_barrier()

    def step(j, carry):
      for kk in range(8):
        sadjb[pl.ds(kk * 16, 16)] = sidx[j, pl.ds(kk * 16, 16)]
        dadjb[pl.ds(kk * 16, 16)] = didx[j, pl.ds(kk * 16, 16)]
      c0 = pltpu.async_copy(as0_hbm.at[sadjb], a0b, sem)
      c1 = pltpu.async_copy(as1_hbm.at[sadjb], a1b, sem)
      c2 = pltpu.async_copy(ad0_hbm.at[dadjb], d0b, sem)
      c3 = pltpu.async_copy(ad1_hbm.at[dadjb], d1b, sem)
      c0.wait(); c1.wait(); c2.wait(); c3.wait()
      for kk in range(8):
        wvs = []
        for h, mh, ab, db in ((0, mh0, a0b, d0b), (1, mh1, a1b, d1b)):
          e = ab[pl.ds(kk * 16, 16)] + db[pl.ds(kk * 16, 16)]
          e = jnp.maximum(e, 0.2 * e)
          wv = jnp.exp(e - mh)
          wbuf[h, pl.ds(kk * 16, 16)] = wv
          wvs.append(wv)
        for i in range(16):
          dbuf[kk * 16 + i, pl.ds(0, 16)] = oh0 * wvs[0][i] + oh1 * wvs[1][i]
      pltpu.sync_copy(wbuf.at[0], w_hbm.at[wid, 0, j])
      pltpu.sync_copy(wbuf.at[1], w_hbm.at[wid, 1, j])
      pltpu.sync_copy(dbuf, slab.at[dadjb], add=True)
      return carry

    lax.fori_loop(0, S, step, 0)
    plsc.subcore_barrier()
    pltpu.sync_copy(slab.at[pl.ds(base, RPT)],
                    den_hbm.at[cid, pl.ds(base, RPT)])

  return k(srcp, dstp, tabs[0], tabs[1], tabs[2], tabs[3], mmax)


def _sc_agg(table_flat, srcp, dstp, w=None, chunks=8):
  weighted = w is not None
  scratch = [
      pltpu.VMEM((S, B), jnp.int32),
      pltpu.VMEM((S, B), jnp.int32),
      pltpu.VMEM((B,), jnp.int32),
      pltpu.VMEM((B,), jnp.int32),
      pltpu.VMEM((B, 128), jnp.float32),
      pltpu.SemaphoreType.DMA,
  ]
  if weighted:
    scratch.insert(2, pltpu.VMEM((HEADS, S, B), jnp.float32))
  scratch.append(pltpu.VMEM_SHARED((NPAD, 128), jnp.float32))

  def body(*refs):
    if weighted:
      (table_hbm, src_hbm, dst_hbm, w_hbm, out_hbm,
       sidx, didx, wv, sadj, dadj, gbuf, sem, slab) = refs
    else:
      (table_hbm, src_hbm, dst_hbm, out_hbm,
       sidx, didx, sadj, dadj, gbuf, sem, slab) = refs
    cid = lax.axis_index("c")
    sid = lax.axis_index("s")
    wid = sid * 2 + cid
    base = sid * RPT
    pltpu.sync_copy(src_hbm.at[wid], sidx)
    pltpu.sync_copy(dst_hbm.at[wid], didx)
    if weighted:
      pltpu.sync_copy(w_hbm.at[wid], wv)
    zv = jnp.zeros((16,), jnp.float32)

    def zrow(r, carry):
      for kk in range(8):
        gbuf[r, pl.ds(kk * 16, 16)] = zv
      return carry

    def chunk(c, carry):
      lax.fori_loop(0, B, zrow, 0)
      for t in range(RPT // B):
        pltpu.sync_copy(gbuf, slab.at[pl.ds(base + t * B, B)])
      plsc.subcore_barrier()

      def step(j, carry2):
        for kk in range(8):
          sadj[pl.ds(kk * 16, 16)] = sidx[j, pl.ds(kk * 16, 16)] + c * N
          dadj[pl.ds(kk * 16, 16)] = didx[j, pl.ds(kk * 16, 16)]
        pltpu.async_copy(table_hbm.at[sadj], gbuf, sem).wait()
        if weighted:
          h = c // (chunks // HEADS)
          for g in range(8):
            wg = wv[h, j, pl.ds(g * 16, 16)]
            for i in range(16):
              r = g * 16 + i
              wb = jnp.full((16,), wg[i], jnp.float32)
              for kk in range(8):
                gbuf[r, pl.ds(kk * 16, 16)] = (
                    gbuf[r, pl.ds(kk * 16, 16)] * wb)
        pltpu.sync_copy(gbuf, slab.at[dadj], add=True)
        return carry2

      lax.fori_loop(0, S, step, 0)
      plsc.subcore_barrier()
      pltpu.sync_copy(slab.at[pl.ds(base, RPT)],
                      out_hbm.at[cid, c, pl.ds(base, RPT)])
      plsc.subcore_barrier()
      return carry

    lax.fori_loop(0, chunks, chunk, 0)

  out_type = jax.ShapeDtypeStruct((2, chunks, NPAD, 128), jnp.float32)
  kfn = pl.kernel(body, out_type=out_type, mesh=_mesh(), scratch_types=scratch)
  if weighted:
    return kfn(table_flat, srcp, dstp, w)
  return kfn(table_flat, srcp, dstp)


def _att_mat(att_src, att_dst):
  z = jnp.zeros((D_IN,), jnp.float32)
  cols = jnp.stack([
      jnp.concatenate([att_src[0], z]),
      jnp.concatenate([z, att_src[1]]),
      jnp.concatenate([att_dst[0], z]),
      jnp.concatenate([z, att_dst[1]]),
  ], axis=1)
  return jnp.pad(cols, ((0, 0), (0, 124)))


def kernel(x, edge_index, W_gat1, att1_src, att1_dst, b_gat1, W_gcn1, b_gcn1,
           W_a, b_a, W_l, b_l, W_d1, b_d1, W_d2, b_d2, W_gcn2, b_gcn2,
           W_gat2, att2_src, att2_dst, b_gat2):
  loop = jnp.arange(N, dtype=jnp.int32)
  src = jnp.concatenate([edge_index[0], loop])
  dst = jnp.concatenate([edge_index[1], loop])
  npad = EPAD - src.shape[0]
  pad_ar = jnp.arange(npad, dtype=jnp.int32)
  src = jnp.concatenate([src, pad_ar % N]).reshape(NW, S, B)
  dst = jnp.concatenate([dst, N + pad_ar % (NPAD - N)]).reshape(NW, S, B)

  deg = _sc_deg(dst)

  def gat(h_in, W, att_s, att_d, b, relu):
    hc = _mm(h_in, W, chunked=True)
    asad = _asad(hc, _att_mat(att_s, att_d))
    m = _colmax(asad)
    asad_t = _transpose128(asad)
    tabs = tuple(asad_t[h] for h in range(4))
    w, den = _sc_gat_w(src, dst, tabs, m)
    p = _sc_agg(hc.reshape(8 * N, 128), src, dst, w=w, chunks=8)
    return _combine_gat(p, den, b, relu)

  def gcn(h_in, W, b):
    g = _mm(h_in, W, deg=deg, chunked=True)
    p = _sc_agg(g.reshape(4 * N, 128), src, dst, chunks=4)
    return _combine_gcn(p, deg, b, relu=True)

  h = gat(x, W_gat1, att1_src, att1_dst, b_gat1, relu=True)
  h = gcn(h, W_gcn1, b_gcn1)
  h = _mm(h, W_a, b=b_a, relu=True)
  z = _mm(h, W_l, b=b_l)
  d = _mm(z, W_d1, b=b_d1, relu=True)
  d = _mm(d, W_d2, b=b_d2, relu=True)
  d = gcn(d, W_gcn2, b_gcn2)
  r = gat(d, W_gat2, att2_src, att2_dst, b_gat2, relu=False)
  return _cdist(r)
